# baseline (speedup 1.0000x reference)
import jax
import jax.numpy as jnp
from jax.experimental import pallas as pl
from jax.experimental.pallas import tpu as pltpu

_TM = 512


def _stage1_body(gl_ref, gh_ref, xl_ref, xh_ref, w1_ref, b1_ref, w2_ref,
                 b2_ref, p2_ref):
    gl = gl_ref[...]
    gh = gh_ref[...]
    m1 = (jnp.dot(gl, xl_ref[...], preferred_element_type=jnp.float32)
          + jnp.dot(gh, xh_ref[...], preferred_element_type=jnp.float32))
    rs = (jnp.sum(gl, axis=1, keepdims=True)
          + jnp.sum(gh, axis=1, keepdims=True))
    h = jnp.dot(m1, w1_ref[...], preferred_element_type=jnp.float32)
    h = jnp.maximum(h + rs * b1_ref[...], 0.0)
    p2 = jnp.dot(h, w2_ref[...], preferred_element_type=jnp.float32)
    p2_ref[...] = p2 + b2_ref[...]


def _stage2_body(gl_ref, gh_ref, pl_ref, ph_ref, o_ref):
    o_ref[...] = (
        jnp.dot(gl_ref[...], pl_ref[...], preferred_element_type=jnp.float32)
        + jnp.dot(gh_ref[...], ph_ref[...],
                  preferred_element_type=jnp.float32))


def kernel(x, G, w1, b1, w2, b2):
    N, C = x.shape
    H = w1.shape[1]
    K = w2.shape[1]
    tm = _TM
    nh = N // 2
    b1r = b1.reshape(1, H)
    b2r = b2.reshape(1, K)

    p2 = pl.pallas_call(
        _stage1_body,
        out_shape=jax.ShapeDtypeStruct((N, K), jnp.float32),
        grid=(N // tm,),
        in_specs=[
            pl.BlockSpec((tm, nh), lambda i: (i, 0)),
            pl.BlockSpec((tm, nh), lambda i: (i, 1)),
            pl.BlockSpec((nh, C), lambda i: (0, 0)),
            pl.BlockSpec((nh, C), lambda i: (1, 0)),
            pl.BlockSpec((C, H), lambda i: (0, 0)),
            pl.BlockSpec((1, H), lambda i: (0, 0)),
            pl.BlockSpec((H, K), lambda i: (0, 0)),
            pl.BlockSpec((1, K), lambda i: (0, 0)),
        ],
        out_specs=pl.BlockSpec((tm, K), lambda i: (i, 0)),
        compiler_params=pltpu.CompilerParams(
            dimension_semantics=("parallel",),
            vmem_limit_bytes=48 * 1024 * 1024,
        ),
    )(G, G, x, x, w1, b1r, w2, b2r)

    out = pl.pallas_call(
        _stage2_body,
        out_shape=jax.ShapeDtypeStruct((N, K), jnp.float32),
        grid=(N // tm,),
        in_specs=[
            pl.BlockSpec((tm, nh), lambda i: (i, 0)),
            pl.BlockSpec((tm, nh), lambda i: (i, 1)),
            pl.BlockSpec((nh, K), lambda i: (0, 0)),
            pl.BlockSpec((nh, K), lambda i: (1, 0)),
        ],
        out_specs=pl.BlockSpec((tm, K), lambda i: (i, 0)),
        compiler_params=pltpu.CompilerParams(
            dimension_semantics=("parallel",),
            vmem_limit_bytes=48 * 1024 * 1024,
        ),
    )(G, G, p2, p2)

    return out
